# baseline (speedup 1.0000x reference)
import jax
import jax.numpy as jnp
from jax.experimental import pallas as pl
from jax.experimental.pallas import tpu as pltpu

B, S, D, H = 2, 2048, 1024, 16
DH = D // H
LOCAL_BASE = 128
FUTURE_BASE = 64
THRESH = 0.5
NEG = -1e9

BQ = 128
BS = 256


NC = 8
CW = D // NC


def _stats_part_kernel(x_ref, st_ref, fq_ref):
    c = pl.program_id(1)
    x = x_ref[0]
    f32 = jnp.float32

    def sdiff_sum(s, scale):
        d = jnp.abs(x[s:] - x[:-s]) * (1.0 / scale)
        m = jnp.sum(d, axis=-1, keepdims=True)
        return jnp.concatenate([m, jnp.zeros((s, 1), f32)], axis=0)

    p = [sdiff_sum(1, 1.0), sdiff_sum(2, 2.0), sdiff_sum(3, 3.0),
         sdiff_sum(4, 4.0), sdiff_sum(5, 5.0)]

    xmean = jnp.mean(x, axis=0, keepdims=True)
    tr = jnp.sum(jnp.abs(x[1:] - xmean), axis=-1, keepdims=True)
    p.append(jnp.concatenate([tr, jnp.zeros((1, 1), f32)], axis=0))
    p.append(jnp.sum(x * x, axis=-1, keepdims=True))

    r0 = x[0:1]
    rl = x[S - 1:S]
    sh = (
        jnp.concatenate([r0, r0, x[:-2]], axis=0),
        jnp.concatenate([r0, x[:-1]], axis=0),
        x,
        jnp.concatenate([x[1:], rl], axis=0),
        jnp.concatenate([x[2:], rl, rl], axis=0),
    )
    m5 = (sh[0] + sh[1] + sh[2] + sh[3] + sh[4]) * 0.2
    var5 = ((sh[0] - m5) ** 2 + (sh[1] - m5) ** 2 + (sh[2] - m5) ** 2
            + (sh[3] - m5) ** 2 + (sh[4] - m5) ** 2) * 0.25
    p.append(jnp.sum(var5, axis=-1, keepdims=True))
    part = jnp.concatenate(p, axis=1)

    dif = x[1:] - x[:-1]
    dmean = jnp.mean(dif, axis=0, keepdims=True)
    fqp = jnp.reshape(jnp.sum((dif - dmean) ** 2) / (S - 2), (1, 1))

    @pl.when(c == 0)
    def _():
        st_ref[0] = part
        fq_ref[0] = fqp

    @pl.when(c != 0)
    def _():
        st_ref[0] = st_ref[0] + part
        fq_ref[0] = fq_ref[0] + fqp


def _stats_post_kernel(st_ref, fq_ref, lw_ref, fw_ref, kp_ref, gc_ref,
                       kpany_ref):
    st = st_ref[0]
    inv_d = 1.0 / D
    d1 = st[:, 0:1] * inv_d
    d2 = st[:, 1:2] * inv_d
    d3 = st[:, 2:3] * inv_d
    d4 = st[:, 3:4] * inv_d
    d5 = st[:, 4:5] * inv_d
    tr = st[:, 5:6] * inv_d
    mag = jnp.sqrt(st[:, 6:7])
    lv = st[:, 7:8] * inv_d
    freq = fq_ref[0, 0, 0] * inv_d

    def norm01(v):
        mn = jnp.min(v)
        mx = jnp.max(v)
        return (v - mn) / (mx - mn + 1e-6)

    imp = norm01(0.5 * d1 + 0.3 * d2 + 0.2 * d4)
    lw_ref[0] = jnp.clip(jnp.round(LOCAL_BASE * (0.5 + 0.5 * imp)), 2,
                         min(S, 2 * LOCAL_BASE)).astype(jnp.int32)

    ti = norm01(tr)
    fw_ref[0] = jnp.clip(jnp.round(FUTURE_BASE * (0.5 + 0.5 * ti)), 1,
                         min(S // 2, FUTURE_BASE)).astype(jnp.int32)

    dmm = 0.4 * d1 + 0.3 * d2 + 0.2 * d3 + 0.1 * d5
    mu = jnp.mean(dmm)
    sd = jnp.sqrt(jnp.sum((dmm - mu) ** 2) / (S - 1))
    thr = mu + THRESH * sd
    gt = (dmm > thr).astype(jnp.int32)
    interior = ((dmm[1:-1] > dmm[:-2]) & (dmm[1:-1] > dmm[2:])).astype(
        jnp.int32) * gt[1:-1]
    kp = jnp.concatenate([gt[0:1], interior, gt[-1:]], axis=0)
    kp_ref[0] = kp
    kpany_ref[0] = jnp.max(kp, keepdims=True)

    fs = lv / (freq + 1e-6)
    sidx = jax.lax.broadcasted_iota(jnp.int32, (S, 1), 0)
    fs = jnp.where((sidx > 0) & (sidx < S - 1), fs, 0.0)
    imp2 = norm01(0.3 * mag + 0.4 * d1 + 0.3 * fs)

    seglen = S // 4
    iota_seg = jax.lax.broadcasted_iota(jnp.int32, (seglen, 1), 0)
    for si in range(4):
        seg = imp2[si * seglen:(si + 1) * seglen]
        taken = jnp.zeros((seglen, 1), jnp.bool_)
        for _ in range(4):
            cur = jnp.where(taken, -1e30, seg)
            mx = jnp.max(cur)
            ismx = (cur == mx) & (~taken)
            idx = jnp.min(jnp.where(ismx, iota_seg, seglen))
            taken = taken | (iota_seg == idx)
        gc_ref[0, si * seglen:(si + 1) * seglen] = taken.astype(jnp.int32)


def _proj_kernel(xq_ref, xk_ref, xv_ref,
                 Wq_ref, bq_ref, Wk_ref, bk_ref, Wv_ref, bv_ref,
                 Wg_ref, bg_ref,
                 gq_ref, bqn_ref, gk_ref, bkn_ref, gv_ref, bvn_ref,
                 qo_ref, ko_ref, vo_ref, go_ref):
    def ln(x, g, b):
        mu = jnp.mean(x, axis=-1, keepdims=True)
        var = jnp.mean((x - mu) ** 2, axis=-1, keepdims=True)
        return (x - mu) / jnp.sqrt(var + 1e-5) * g + b

    def matmul_t(x, w_ref, b):
        return jax.lax.dot_general(
            x, w_ref[...], (((1,), (1,)), ((), ())),
            preferred_element_type=jnp.float32) + b

    xq = xq_ref[0]
    qo_ref[0] = matmul_t(ln(xq, gq_ref[...], bqn_ref[...]), Wq_ref, bq_ref[...])
    ko_ref[0] = matmul_t(ln(xk_ref[0], gk_ref[...], bkn_ref[...]), Wk_ref,
                         bk_ref[...])
    vo_ref[0] = matmul_t(ln(xv_ref[0], gv_ref[...], bvn_ref[...]), Wv_ref,
                         bv_ref[...])
    go_ref[0] = jax.nn.sigmoid(matmul_t(xq, Wg_ref, bg_ref[...]))


def _attn_kernel(q_ref, k_ref, v_ref, g_ref,
                 lw_ref, fw_ref, kpr_ref, kpc_ref, gcc_ref, kpany_ref,
                 Wo_ref, bo_ref, o_ref):
    qi = pl.program_id(1)
    qblk = q_ref[0]
    kblk = k_ref[0]
    vblk = v_ref[0]

    i = qi * BQ + jax.lax.broadcasted_iota(jnp.int32, (BQ, S), 0)
    j = jax.lax.broadcasted_iota(jnp.int32, (BQ, S), 1)
    diff = i - j
    lwr = lw_ref[0]
    fwr = fw_ref[0]
    win = ((diff >= 0) & (diff <= lwr)) | ((diff < 0) & (-diff <= fwr))
    kpr = kpr_ref[0] > 0
    kpc = kpc_ref[0] > 0
    gcc = gcc_ref[0] > 0
    kpany = kpany_ref[0, 0, 0] > 0
    jrow = j[0:1]
    fb = ((jrow == 0) | (jrow == (S - 1) // 4) | (jrow == (S - 1) // 2)
          | (jrow == (3 * (S - 1)) // 4) | (jrow == S - 1))
    kpm = (kpany & (kpr | kpc)) | ((~kpany) & fb)
    mask = win | kpm | gcc
    negbias = jnp.where(mask, 0.0, NEG)

    ctxs = []
    for h in range(H):
        qh = qblk[:, h * DH:(h + 1) * DH]
        kh = kblk[:, h * DH:(h + 1) * DH]
        vh = vblk[:, h * DH:(h + 1) * DH]
        s = jax.lax.dot_general(qh, kh, (((1,), (1,)), ((), ())),
                                preferred_element_type=jnp.float32) * 0.125
        s = s + negbias
        m = jnp.max(s, axis=-1, keepdims=True)
        p = jnp.exp(s - m)
        inv = 1.0 / jnp.sum(p, axis=-1, keepdims=True)
        ctxs.append(jnp.dot(p, vh, preferred_element_type=jnp.float32) * inv)
    ctx = jnp.concatenate(ctxs, axis=-1)
    ctxg = ctx * g_ref[0]
    o_ref[0] = jax.lax.dot_general(
        ctxg, Wo_ref[...], (((1,), (1,)), ((), ())),
        preferred_element_type=jnp.float32) + bo_ref[...]


def _mask_ingredients(queries):
    f32 = jnp.float32
    i32 = jnp.int32

    stats, fq = pl.pallas_call(
        _stats_part_kernel,
        grid=(B, NC),
        in_specs=[pl.BlockSpec((1, S, CW), lambda b, c: (b, 0, c))],
        out_specs=[
            pl.BlockSpec((1, S, 8), lambda b, c: (b, 0, 0)),
            pl.BlockSpec((1, 1, 1), lambda b, c: (b, 0, 0)),
        ],
        out_shape=[
            jax.ShapeDtypeStruct((B, S, 8), f32),
            jax.ShapeDtypeStruct((B, 1, 1), f32),
        ],
    )(queries)

    lw, fw, kp, gc, kpany = pl.pallas_call(
        _stats_post_kernel,
        grid=(B,),
        in_specs=[pl.BlockSpec((1, S, 8), lambda b: (b, 0, 0)),
                  pl.BlockSpec((1, 1, 1), lambda b: (b, 0, 0))],
        out_specs=[
            pl.BlockSpec((1, S, 1), lambda b: (b, 0, 0)),
            pl.BlockSpec((1, S, 1), lambda b: (b, 0, 0)),
            pl.BlockSpec((1, S, 1), lambda b: (b, 0, 0)),
            pl.BlockSpec((1, S, 1), lambda b: (b, 0, 0)),
            pl.BlockSpec((1, 1, 1), lambda b: (b, 0, 0)),
        ],
        out_shape=[
            jax.ShapeDtypeStruct((B, S, 1), i32),
            jax.ShapeDtypeStruct((B, S, 1), i32),
            jax.ShapeDtypeStruct((B, S, 1), i32),
            jax.ShapeDtypeStruct((B, S, 1), i32),
            jax.ShapeDtypeStruct((B, 1, 1), i32),
        ],
    )(stats, fq)
    return lw, fw, kp, gc, kpany


def kernel(queries, keys, values, Wq, bq, Wk, bk, Wv, bv, Wo, bo, Wg, bg,
           g_q, b_q, g_k, b_k, g_v, b_v):
    f32 = jnp.float32

    lw, fw, kp, gc, kpany = _mask_ingredients(queries)
    kp_col = kp.reshape(B, 1, S)
    gc_col = gc.reshape(B, 1, S)

    b2 = lambda a: a.reshape(1, D)
    nrow = S // BS
    wspec = pl.BlockSpec((D, D), lambda b, r: (0, 0))
    bspec = pl.BlockSpec((1, D), lambda b, r: (0, 0))
    xspec = pl.BlockSpec((1, BS, D), lambda b, r: (b, r, 0))
    q, k, v, gate = pl.pallas_call(
        _proj_kernel,
        grid=(B, nrow),
        in_specs=[xspec, xspec, xspec,
                  wspec, bspec, wspec, bspec, wspec, bspec, wspec, bspec,
                  bspec, bspec, bspec, bspec, bspec, bspec],
        out_specs=[xspec, xspec, xspec, xspec],
        out_shape=[jax.ShapeDtypeStruct((B, S, D), f32)] * 4,
    )(queries, keys, values,
      Wq, b2(bq), Wk, b2(bk), Wv, b2(bv), Wg, b2(bg),
      b2(g_q), b2(b_q), b2(g_k), b2(b_k), b2(g_v), b2(b_v))

    nq = S // BQ
    qspec = pl.BlockSpec((1, BQ, D), lambda b, r: (b, r, 0))
    fullspec = pl.BlockSpec((1, S, D), lambda b, r: (b, 0, 0))
    rowspec = pl.BlockSpec((1, BQ, 1), lambda b, r: (b, r, 0))
    colspec = pl.BlockSpec((1, 1, S), lambda b, r: (b, 0, 0))
    out = pl.pallas_call(
        _attn_kernel,
        grid=(B, nq),
        in_specs=[qspec, fullspec, fullspec, qspec,
                  rowspec, rowspec, rowspec, colspec, colspec,
                  pl.BlockSpec((1, 1, 1), lambda b, r: (b, 0, 0)),
                  pl.BlockSpec((D, D), lambda b, r: (0, 0)),
                  pl.BlockSpec((1, D), lambda b, r: (0, 0))],
        out_specs=qspec,
        out_shape=jax.ShapeDtypeStruct((B, S, D), f32),
        compiler_params=pltpu.CompilerParams(
            vmem_limit_bytes=64 * 1024 * 1024),
    )(q, k, v, gate, lw, fw, kp, kp_col, gc_col, kpany, Wo, b2(bo))
    return out


# speedup vs baseline: 1.1905x; 1.1905x over previous
import jax
import jax.numpy as jnp
from jax.experimental import pallas as pl
from jax.experimental.pallas import tpu as pltpu

B, S, D, H = 2, 2048, 1024, 16
DH = D // H
LOCAL_BASE = 128
FUTURE_BASE = 64
THRESH = 0.5
NEG = -1e9

BQ = 256
BS = 256


NC = 8
CW = D // NC


def _stats_part_kernel(x_ref, st_ref, fq_ref):
    c = pl.program_id(1)
    x = x_ref[0]
    f32 = jnp.float32

    def sdiff_sum(s, scale):
        d = jnp.abs(x[s:] - x[:-s]) * (1.0 / scale)
        m = jnp.sum(d, axis=-1, keepdims=True)
        return jnp.concatenate([m, jnp.zeros((s, 1), f32)], axis=0)

    p = [sdiff_sum(1, 1.0), sdiff_sum(2, 2.0), sdiff_sum(3, 3.0),
         sdiff_sum(4, 4.0), sdiff_sum(5, 5.0)]

    xmean = jnp.mean(x, axis=0, keepdims=True)
    tr = jnp.sum(jnp.abs(x[1:] - xmean), axis=-1, keepdims=True)
    p.append(jnp.concatenate([tr, jnp.zeros((1, 1), f32)], axis=0))
    p.append(jnp.sum(x * x, axis=-1, keepdims=True))

    r0 = x[0:1]
    rl = x[S - 1:S]
    sh = (
        jnp.concatenate([r0, r0, x[:-2]], axis=0),
        jnp.concatenate([r0, x[:-1]], axis=0),
        x,
        jnp.concatenate([x[1:], rl], axis=0),
        jnp.concatenate([x[2:], rl, rl], axis=0),
    )
    m5 = (sh[0] + sh[1] + sh[2] + sh[3] + sh[4]) * 0.2
    var5 = ((sh[0] - m5) ** 2 + (sh[1] - m5) ** 2 + (sh[2] - m5) ** 2
            + (sh[3] - m5) ** 2 + (sh[4] - m5) ** 2) * 0.25
    p.append(jnp.sum(var5, axis=-1, keepdims=True))
    part = jnp.concatenate(p, axis=1)

    dif = x[1:] - x[:-1]
    dmean = jnp.mean(dif, axis=0, keepdims=True)
    fqp = jnp.reshape(jnp.sum((dif - dmean) ** 2) / (S - 2), (1, 1))

    @pl.when(c == 0)
    def _():
        st_ref[0] = part
        fq_ref[0] = fqp

    @pl.when(c != 0)
    def _():
        st_ref[0] = st_ref[0] + part
        fq_ref[0] = fq_ref[0] + fqp


def _stats_post_kernel(st_ref, fq_ref, lw_ref, fw_ref, kp_ref, gc_ref,
                       kpany_ref):
    st = st_ref[0]
    inv_d = 1.0 / D
    d1 = st[:, 0:1] * inv_d
    d2 = st[:, 1:2] * inv_d
    d3 = st[:, 2:3] * inv_d
    d4 = st[:, 3:4] * inv_d
    d5 = st[:, 4:5] * inv_d
    tr = st[:, 5:6] * inv_d
    mag = jnp.sqrt(st[:, 6:7])
    lv = st[:, 7:8] * inv_d
    freq = fq_ref[0, 0, 0] * inv_d

    def norm01(v):
        mn = jnp.min(v)
        mx = jnp.max(v)
        return (v - mn) / (mx - mn + 1e-6)

    imp = norm01(0.5 * d1 + 0.3 * d2 + 0.2 * d4)
    lw_ref[0] = jnp.clip(jnp.round(LOCAL_BASE * (0.5 + 0.5 * imp)), 2,
                         min(S, 2 * LOCAL_BASE)).astype(jnp.int32)

    ti = norm01(tr)
    fw_ref[0] = jnp.clip(jnp.round(FUTURE_BASE * (0.5 + 0.5 * ti)), 1,
                         min(S // 2, FUTURE_BASE)).astype(jnp.int32)

    dmm = 0.4 * d1 + 0.3 * d2 + 0.2 * d3 + 0.1 * d5
    mu = jnp.mean(dmm)
    sd = jnp.sqrt(jnp.sum((dmm - mu) ** 2) / (S - 1))
    thr = mu + THRESH * sd
    gt = (dmm > thr).astype(jnp.int32)
    interior = ((dmm[1:-1] > dmm[:-2]) & (dmm[1:-1] > dmm[2:])).astype(
        jnp.int32) * gt[1:-1]
    kp = jnp.concatenate([gt[0:1], interior, gt[-1:]], axis=0)
    kp_ref[0] = kp
    kpany_ref[0] = jnp.max(kp, keepdims=True)

    fs = lv / (freq + 1e-6)
    sidx = jax.lax.broadcasted_iota(jnp.int32, (S, 1), 0)
    fs = jnp.where((sidx > 0) & (sidx < S - 1), fs, 0.0)
    imp2 = norm01(0.3 * mag + 0.4 * d1 + 0.3 * fs)

    seglen = S // 4
    iota_seg = jax.lax.broadcasted_iota(jnp.int32, (seglen, 1), 0)
    for si in range(4):
        seg = imp2[si * seglen:(si + 1) * seglen]
        taken = jnp.zeros((seglen, 1), jnp.bool_)
        for _ in range(4):
            cur = jnp.where(taken, -1e30, seg)
            mx = jnp.max(cur)
            ismx = (cur == mx) & (~taken)
            idx = jnp.min(jnp.where(ismx, iota_seg, seglen))
            taken = taken | (iota_seg == idx)
        gc_ref[0, si * seglen:(si + 1) * seglen] = taken.astype(jnp.int32)


def _proj_kernel(xq_ref, xk_ref, xv_ref,
                 Wq_ref, bq_ref, Wk_ref, bk_ref, Wv_ref, bv_ref,
                 Wg_ref, bg_ref,
                 gq_ref, bqn_ref, gk_ref, bkn_ref, gv_ref, bvn_ref,
                 qo_ref, ko_ref, vo_ref, go_ref):
    def ln(x, g, b):
        mu = jnp.mean(x, axis=-1, keepdims=True)
        var = jnp.mean((x - mu) ** 2, axis=-1, keepdims=True)
        return (x - mu) / jnp.sqrt(var + 1e-5) * g + b

    def matmul_t(x, w_ref, b):
        return jax.lax.dot_general(
            x, w_ref[...], (((1,), (1,)), ((), ())),
            preferred_element_type=jnp.float32) + b

    xq = xq_ref[0]
    qo_ref[0] = matmul_t(ln(xq, gq_ref[...], bqn_ref[...]), Wq_ref, bq_ref[...])
    ko_ref[0] = matmul_t(ln(xk_ref[0], gk_ref[...], bkn_ref[...]), Wk_ref,
                         bk_ref[...])
    vo_ref[0] = matmul_t(ln(xv_ref[0], gv_ref[...], bvn_ref[...]), Wv_ref,
                         bv_ref[...])
    go_ref[0] = jax.nn.sigmoid(matmul_t(xq, Wg_ref, bg_ref[...]))


def _attn_kernel(q_ref, k_ref, v_ref, g_ref,
                 lw_ref, fw_ref, kpr_ref, kpc_ref, gcc_ref, kpany_ref,
                 Wo_ref, bo_ref, o_ref, s_scr, ctx_scr):
    qi = pl.program_id(1)
    qblk = q_ref[0]
    kblk = k_ref[0]
    vblk = v_ref[0]

    icol = qi * BQ + jax.lax.broadcasted_iota(jnp.int32, (BQ, 1), 0)
    jrow = jax.lax.broadcasted_iota(jnp.int32, (1, S), 1)
    lwr = lw_ref[0]
    fwr = fw_ref[0]
    kpr = kpr_ref[0] > 0
    kpc = kpc_ref[0] > 0
    gcc = gcc_ref[0] > 0
    kpany = kpany_ref[0, 0, 0] > 0
    fb = ((jrow == 0) | (jrow == (S - 1) // 4) | (jrow == (S - 1) // 2)
          | (jrow == (3 * (S - 1)) // 4) | (jrow == S - 1))
    win = (jrow >= icol - lwr) & (jrow <= icol + fwr)
    colterm = (kpany & kpc) | ((~kpany) & fb) | gcc
    rowterm = kpany & kpr
    mask = (win | rowterm) | colterm
    negbias = jnp.where(mask, 0.0, NEG)

    for h in range(H):
        qh = qblk[:, h * DH:(h + 1) * DH]
        kh = kblk[:, h * DH:(h + 1) * DH]
        vh = vblk[:, h * DH:(h + 1) * DH]
        s_scr[...] = jax.lax.dot_general(
            qh, kh, (((1,), (1,)), ((), ())),
            preferred_element_type=jnp.float32) * 0.125 + negbias
        s = s_scr[...]
        m = jnp.max(s, axis=-1, keepdims=True)
        p = jnp.exp(s - m)
        p = p / jnp.sum(p, axis=-1, keepdims=True)
        ctx_scr[:, h * DH:(h + 1) * DH] = jnp.dot(
            p, vh, preferred_element_type=jnp.float32)
    ctxg = ctx_scr[...] * g_ref[0]
    o_ref[0] = jax.lax.dot_general(
        ctxg, Wo_ref[...], (((1,), (1,)), ((), ())),
        preferred_element_type=jnp.float32) + bo_ref[...]


def _mask_ingredients(queries):
    f32 = jnp.float32
    i32 = jnp.int32

    stats, fq = pl.pallas_call(
        _stats_part_kernel,
        grid=(B, NC),
        in_specs=[pl.BlockSpec((1, S, CW), lambda b, c: (b, 0, c))],
        out_specs=[
            pl.BlockSpec((1, S, 8), lambda b, c: (b, 0, 0)),
            pl.BlockSpec((1, 1, 1), lambda b, c: (b, 0, 0)),
        ],
        out_shape=[
            jax.ShapeDtypeStruct((B, S, 8), f32),
            jax.ShapeDtypeStruct((B, 1, 1), f32),
        ],
    )(queries)

    lw, fw, kp, gc, kpany = pl.pallas_call(
        _stats_post_kernel,
        grid=(B,),
        in_specs=[pl.BlockSpec((1, S, 8), lambda b: (b, 0, 0)),
                  pl.BlockSpec((1, 1, 1), lambda b: (b, 0, 0))],
        out_specs=[
            pl.BlockSpec((1, S, 1), lambda b: (b, 0, 0)),
            pl.BlockSpec((1, S, 1), lambda b: (b, 0, 0)),
            pl.BlockSpec((1, S, 1), lambda b: (b, 0, 0)),
            pl.BlockSpec((1, S, 1), lambda b: (b, 0, 0)),
            pl.BlockSpec((1, 1, 1), lambda b: (b, 0, 0)),
        ],
        out_shape=[
            jax.ShapeDtypeStruct((B, S, 1), i32),
            jax.ShapeDtypeStruct((B, S, 1), i32),
            jax.ShapeDtypeStruct((B, S, 1), i32),
            jax.ShapeDtypeStruct((B, S, 1), i32),
            jax.ShapeDtypeStruct((B, 1, 1), i32),
        ],
    )(stats, fq)
    return lw, fw, kp, gc, kpany


def kernel(queries, keys, values, Wq, bq, Wk, bk, Wv, bv, Wo, bo, Wg, bg,
           g_q, b_q, g_k, b_k, g_v, b_v):
    f32 = jnp.float32

    lw, fw, kp, gc, kpany = _mask_ingredients(queries)
    kp_col = kp.reshape(B, 1, S)
    gc_col = gc.reshape(B, 1, S)

    b2 = lambda a: a.reshape(1, D)
    nrow = S // BS
    wspec = pl.BlockSpec((D, D), lambda b, r: (0, 0))
    bspec = pl.BlockSpec((1, D), lambda b, r: (0, 0))
    xspec = pl.BlockSpec((1, BS, D), lambda b, r: (b, r, 0))
    q, k, v, gate = pl.pallas_call(
        _proj_kernel,
        grid=(B, nrow),
        in_specs=[xspec, xspec, xspec,
                  wspec, bspec, wspec, bspec, wspec, bspec, wspec, bspec,
                  bspec, bspec, bspec, bspec, bspec, bspec],
        out_specs=[xspec, xspec, xspec, xspec],
        out_shape=[jax.ShapeDtypeStruct((B, S, D), f32)] * 4,
    )(queries, keys, values,
      Wq, b2(bq), Wk, b2(bk), Wv, b2(bv), Wg, b2(bg),
      b2(g_q), b2(b_q), b2(g_k), b2(b_k), b2(g_v), b2(b_v))

    nq = S // BQ
    qspec = pl.BlockSpec((1, BQ, D), lambda b, r: (b, r, 0))
    fullspec = pl.BlockSpec((1, S, D), lambda b, r: (b, 0, 0))
    rowspec = pl.BlockSpec((1, BQ, 1), lambda b, r: (b, r, 0))
    colspec = pl.BlockSpec((1, 1, S), lambda b, r: (b, 0, 0))
    out = pl.pallas_call(
        _attn_kernel,
        grid=(B, nq),
        in_specs=[qspec, fullspec, fullspec, qspec,
                  rowspec, rowspec, rowspec, colspec, colspec,
                  pl.BlockSpec((1, 1, 1), lambda b, r: (b, 0, 0)),
                  pl.BlockSpec((D, D), lambda b, r: (0, 0)),
                  pl.BlockSpec((1, D), lambda b, r: (0, 0))],
        out_specs=qspec,
        out_shape=jax.ShapeDtypeStruct((B, S, D), f32),
        scratch_shapes=[pltpu.VMEM((BQ, S), f32), pltpu.VMEM((BQ, D), f32)],
        compiler_params=pltpu.CompilerParams(
            vmem_limit_bytes=64 * 1024 * 1024),
    )(q, k, v, gate, lw, fw, kp, kp_col, gc_col, kpany, Wo, b2(bo))
    return out


# speedup vs baseline: 1.4372x; 1.2073x over previous
import jax
import jax.numpy as jnp
from jax.experimental import pallas as pl
from jax.experimental.pallas import tpu as pltpu

B, S, D, H = 2, 2048, 1024, 16
DH = D // H
LOCAL_BASE = 128
FUTURE_BASE = 64
THRESH = 0.5
NEG = -1e9

BQ = 256
BS = 256


NC = 8
CW = D // NC


def _stats_part_kernel(x_ref, st_ref, fq_ref):
    c = pl.program_id(1)
    x = x_ref[0]
    f32 = jnp.float32

    def sdiff_sum(s, scale):
        d = jnp.abs(x[s:] - x[:-s]) * (1.0 / scale)
        m = jnp.sum(d, axis=-1, keepdims=True)
        return jnp.concatenate([m, jnp.zeros((s, 1), f32)], axis=0)

    p = [sdiff_sum(1, 1.0), sdiff_sum(2, 2.0), sdiff_sum(3, 3.0),
         sdiff_sum(4, 4.0), sdiff_sum(5, 5.0)]

    xmean = jnp.mean(x, axis=0, keepdims=True)
    tr = jnp.sum(jnp.abs(x[1:] - xmean), axis=-1, keepdims=True)
    p.append(jnp.concatenate([tr, jnp.zeros((1, 1), f32)], axis=0))
    p.append(jnp.sum(x * x, axis=-1, keepdims=True))

    r0 = x[0:1]
    rl = x[S - 1:S]
    sh = (
        jnp.concatenate([r0, r0, x[:-2]], axis=0),
        jnp.concatenate([r0, x[:-1]], axis=0),
        x,
        jnp.concatenate([x[1:], rl], axis=0),
        jnp.concatenate([x[2:], rl, rl], axis=0),
    )
    m5 = (sh[0] + sh[1] + sh[2] + sh[3] + sh[4]) * 0.2
    var5 = ((sh[0] - m5) ** 2 + (sh[1] - m5) ** 2 + (sh[2] - m5) ** 2
            + (sh[3] - m5) ** 2 + (sh[4] - m5) ** 2) * 0.25
    p.append(jnp.sum(var5, axis=-1, keepdims=True))
    part = jnp.concatenate(p, axis=1)

    dif = x[1:] - x[:-1]
    dmean = jnp.mean(dif, axis=0, keepdims=True)
    fqp = jnp.reshape(jnp.sum((dif - dmean) ** 2) / (S - 2), (1, 1))

    @pl.when(c == 0)
    def _():
        st_ref[0] = part
        fq_ref[0] = fqp

    @pl.when(c != 0)
    def _():
        st_ref[0] = st_ref[0] + part
        fq_ref[0] = fq_ref[0] + fqp


def _stats_post_kernel(st_ref, fq_ref, lw_ref, fw_ref, kp_ref, gc_ref,
                       kpany_ref):
    st = st_ref[0]
    inv_d = 1.0 / D
    d1 = st[:, 0:1] * inv_d
    d2 = st[:, 1:2] * inv_d
    d3 = st[:, 2:3] * inv_d
    d4 = st[:, 3:4] * inv_d
    d5 = st[:, 4:5] * inv_d
    tr = st[:, 5:6] * inv_d
    mag = jnp.sqrt(st[:, 6:7])
    lv = st[:, 7:8] * inv_d
    freq = fq_ref[0, 0, 0] * inv_d

    def norm01(v):
        mn = jnp.min(v)
        mx = jnp.max(v)
        return (v - mn) / (mx - mn + 1e-6)

    imp = norm01(0.5 * d1 + 0.3 * d2 + 0.2 * d4)
    lw_ref[0] = jnp.clip(jnp.round(LOCAL_BASE * (0.5 + 0.5 * imp)), 2,
                         min(S, 2 * LOCAL_BASE)).astype(jnp.int32)

    ti = norm01(tr)
    fw_ref[0] = jnp.clip(jnp.round(FUTURE_BASE * (0.5 + 0.5 * ti)), 1,
                         min(S // 2, FUTURE_BASE)).astype(jnp.int32)

    dmm = 0.4 * d1 + 0.3 * d2 + 0.2 * d3 + 0.1 * d5
    mu = jnp.mean(dmm)
    sd = jnp.sqrt(jnp.sum((dmm - mu) ** 2) / (S - 1))
    thr = mu + THRESH * sd
    gt = (dmm > thr).astype(jnp.int32)
    interior = ((dmm[1:-1] > dmm[:-2]) & (dmm[1:-1] > dmm[2:])).astype(
        jnp.int32) * gt[1:-1]
    kp = jnp.concatenate([gt[0:1], interior, gt[-1:]], axis=0)
    kp_ref[0] = kp
    kpany_ref[0] = jnp.max(kp, keepdims=True)

    fs = lv / (freq + 1e-6)
    sidx = jax.lax.broadcasted_iota(jnp.int32, (S, 1), 0)
    fs = jnp.where((sidx > 0) & (sidx < S - 1), fs, 0.0)
    imp2 = norm01(0.3 * mag + 0.4 * d1 + 0.3 * fs)

    seglen = S // 4
    iota_seg = jax.lax.broadcasted_iota(jnp.int32, (seglen, 1), 0)
    for si in range(4):
        seg = imp2[si * seglen:(si + 1) * seglen]
        taken = jnp.zeros((seglen, 1), jnp.bool_)
        for _ in range(4):
            cur = jnp.where(taken, -1e30, seg)
            mx = jnp.max(cur)
            ismx = (cur == mx) & (~taken)
            idx = jnp.min(jnp.where(ismx, iota_seg, seglen))
            taken = taken | (iota_seg == idx)
        gc_ref[0, si * seglen:(si + 1) * seglen] = taken.astype(jnp.int32)


def _proj_kernel(xq_ref, xk_ref, xv_ref,
                 Wq_ref, bq_ref, Wk_ref, bk_ref, Wv_ref, bv_ref,
                 Wg_ref, bg_ref,
                 gq_ref, bqn_ref, gk_ref, bkn_ref, gv_ref, bvn_ref,
                 qo_ref, ko_ref, vo_ref, go_ref):
    def ln(x, g, b):
        mu = jnp.mean(x, axis=-1, keepdims=True)
        var = jnp.mean((x - mu) ** 2, axis=-1, keepdims=True)
        return (x - mu) / jnp.sqrt(var + 1e-5) * g + b

    def matmul_t(x, w_ref, b):
        return jax.lax.dot_general(
            x, w_ref[...], (((1,), (1,)), ((), ())),
            preferred_element_type=jnp.float32) + b

    xq = xq_ref[0]
    qo_ref[0] = matmul_t(ln(xq, gq_ref[...], bqn_ref[...]), Wq_ref, bq_ref[...])
    ko_ref[0] = matmul_t(ln(xk_ref[0], gk_ref[...], bkn_ref[...]), Wk_ref,
                         bk_ref[...])
    vo_ref[0] = matmul_t(ln(xv_ref[0], gv_ref[...], bvn_ref[...]), Wv_ref,
                         bv_ref[...])
    go_ref[0] = jax.nn.sigmoid(matmul_t(xq, Wg_ref, bg_ref[...]))


def _attn_kernel(q_ref, k_ref, v_ref, g_ref,
                 lw_ref, fw_ref, kpr_ref, kpc_ref, gcc_ref, kpany_ref,
                 Wo_ref, bo_ref, o_ref, s_scr, ctx_scr):
    qi = pl.program_id(1)
    qblk = q_ref[0]
    kblk = k_ref[0]
    vblk = v_ref[0]

    icol = qi * BQ + jax.lax.broadcasted_iota(jnp.int32, (BQ, 1), 0)
    jrow = jax.lax.broadcasted_iota(jnp.int32, (1, S), 1)
    lwr = lw_ref[0]
    fwr = fw_ref[0]
    kpr = kpr_ref[0] > 0
    kpc = kpc_ref[0] > 0
    gcc = gcc_ref[0] > 0
    kpany = kpany_ref[0, 0, 0] > 0
    fb = ((jrow == 0) | (jrow == (S - 1) // 4) | (jrow == (S - 1) // 2)
          | (jrow == (3 * (S - 1)) // 4) | (jrow == S - 1))
    win = (jrow >= icol - lwr) & (jrow <= icol + fwr)
    colterm = (kpany & kpc) | ((~kpany) & fb) | gcc
    rowterm = kpany & kpr
    mask = (win | rowterm) | colterm
    negbias = jnp.where(mask, 0.0, NEG)

    for h in range(H):
        qh = qblk[:, h * DH:(h + 1) * DH]
        kh = kblk[:, h * DH:(h + 1) * DH]
        vh = vblk[:, h * DH:(h + 1) * DH]
        s_scr[...] = jax.lax.dot_general(
            qh, kh, (((1,), (1,)), ((), ())),
            preferred_element_type=jnp.float32) * 0.125 + negbias
        p = jnp.exp(s_scr[...] - 40.0)
        p = p / jnp.sum(p, axis=-1, keepdims=True)
        ctx_scr[:, h * DH:(h + 1) * DH] = jnp.dot(
            p, vh, preferred_element_type=jnp.float32)
    ctxg = ctx_scr[...] * g_ref[0]
    o_ref[0] = jax.lax.dot_general(
        ctxg, Wo_ref[...], (((1,), (1,)), ((), ())),
        preferred_element_type=jnp.float32) + bo_ref[...]


def _mask_ingredients(queries):
    f32 = jnp.float32
    i32 = jnp.int32

    stats, fq = pl.pallas_call(
        _stats_part_kernel,
        grid=(B, NC),
        in_specs=[pl.BlockSpec((1, S, CW), lambda b, c: (b, 0, c))],
        out_specs=[
            pl.BlockSpec((1, S, 8), lambda b, c: (b, 0, 0)),
            pl.BlockSpec((1, 1, 1), lambda b, c: (b, 0, 0)),
        ],
        out_shape=[
            jax.ShapeDtypeStruct((B, S, 8), f32),
            jax.ShapeDtypeStruct((B, 1, 1), f32),
        ],
    )(queries)

    lw, fw, kp, gc, kpany = pl.pallas_call(
        _stats_post_kernel,
        grid=(B,),
        in_specs=[pl.BlockSpec((1, S, 8), lambda b: (b, 0, 0)),
                  pl.BlockSpec((1, 1, 1), lambda b: (b, 0, 0))],
        out_specs=[
            pl.BlockSpec((1, S, 1), lambda b: (b, 0, 0)),
            pl.BlockSpec((1, S, 1), lambda b: (b, 0, 0)),
            pl.BlockSpec((1, S, 1), lambda b: (b, 0, 0)),
            pl.BlockSpec((1, S, 1), lambda b: (b, 0, 0)),
            pl.BlockSpec((1, 1, 1), lambda b: (b, 0, 0)),
        ],
        out_shape=[
            jax.ShapeDtypeStruct((B, S, 1), i32),
            jax.ShapeDtypeStruct((B, S, 1), i32),
            jax.ShapeDtypeStruct((B, S, 1), i32),
            jax.ShapeDtypeStruct((B, S, 1), i32),
            jax.ShapeDtypeStruct((B, 1, 1), i32),
        ],
    )(stats, fq)
    return lw, fw, kp, gc, kpany


def kernel(queries, keys, values, Wq, bq, Wk, bk, Wv, bv, Wo, bo, Wg, bg,
           g_q, b_q, g_k, b_k, g_v, b_v):
    f32 = jnp.float32

    lw, fw, kp, gc, kpany = _mask_ingredients(queries)
    kp_col = kp.reshape(B, 1, S)
    gc_col = gc.reshape(B, 1, S)

    b2 = lambda a: a.reshape(1, D)
    nrow = S // BS
    wspec = pl.BlockSpec((D, D), lambda b, r: (0, 0))
    bspec = pl.BlockSpec((1, D), lambda b, r: (0, 0))
    xspec = pl.BlockSpec((1, BS, D), lambda b, r: (b, r, 0))
    q, k, v, gate = pl.pallas_call(
        _proj_kernel,
        grid=(B, nrow),
        in_specs=[xspec, xspec, xspec,
                  wspec, bspec, wspec, bspec, wspec, bspec, wspec, bspec,
                  bspec, bspec, bspec, bspec, bspec, bspec],
        out_specs=[xspec, xspec, xspec, xspec],
        out_shape=[jax.ShapeDtypeStruct((B, S, D), f32)] * 4,
    )(queries, keys, values,
      Wq, b2(bq), Wk, b2(bk), Wv, b2(bv), Wg, b2(bg),
      b2(g_q), b2(b_q), b2(g_k), b2(b_k), b2(g_v), b2(b_v))

    nq = S // BQ
    qspec = pl.BlockSpec((1, BQ, D), lambda b, r: (b, r, 0))
    fullspec = pl.BlockSpec((1, S, D), lambda b, r: (b, 0, 0))
    rowspec = pl.BlockSpec((1, BQ, 1), lambda b, r: (b, r, 0))
    colspec = pl.BlockSpec((1, 1, S), lambda b, r: (b, 0, 0))
    out = pl.pallas_call(
        _attn_kernel,
        grid=(B, nq),
        in_specs=[qspec, fullspec, fullspec, qspec,
                  rowspec, rowspec, rowspec, colspec, colspec,
                  pl.BlockSpec((1, 1, 1), lambda b, r: (b, 0, 0)),
                  pl.BlockSpec((D, D), lambda b, r: (0, 0)),
                  pl.BlockSpec((1, D), lambda b, r: (0, 0))],
        out_specs=qspec,
        out_shape=jax.ShapeDtypeStruct((B, S, D), f32),
        scratch_shapes=[pltpu.VMEM((BQ, S), f32), pltpu.VMEM((BQ, D), f32)],
        compiler_params=pltpu.CompilerParams(
            vmem_limit_bytes=64 * 1024 * 1024),
    )(q, k, v, gate, lw, fw, kp, kp_col, gc_col, kpany, Wo, b2(bo))
    return out


# speedup vs baseline: 1.4823x; 1.0313x over previous
import jax
import jax.numpy as jnp
from jax.experimental import pallas as pl
from jax.experimental.pallas import tpu as pltpu

B, S, D, H = 2, 2048, 1024, 16
DH = D // H
LOCAL_BASE = 128
FUTURE_BASE = 64
THRESH = 0.5
NEG = -1e9

BQ = 256
BS = 256


NC = 8
CW = D // NC


def _stats_part_kernel(x_ref, st_ref, fq_ref):
    c = pl.program_id(1)
    x = x_ref[0]
    f32 = jnp.float32

    def sdiff_sum(s, scale):
        d = jnp.abs(x[s:] - x[:-s]) * (1.0 / scale)
        m = jnp.sum(d, axis=-1, keepdims=True)
        return jnp.concatenate([m, jnp.zeros((s, 1), f32)], axis=0)

    p = [sdiff_sum(1, 1.0), sdiff_sum(2, 2.0), sdiff_sum(3, 3.0),
         sdiff_sum(4, 4.0), sdiff_sum(5, 5.0)]

    xmean = jnp.mean(x, axis=0, keepdims=True)
    tr = jnp.sum(jnp.abs(x[1:] - xmean), axis=-1, keepdims=True)
    p.append(jnp.concatenate([tr, jnp.zeros((1, 1), f32)], axis=0))
    p.append(jnp.sum(x * x, axis=-1, keepdims=True))

    r0 = x[0:1]
    rl = x[S - 1:S]
    sh = (
        jnp.concatenate([r0, r0, x[:-2]], axis=0),
        jnp.concatenate([r0, x[:-1]], axis=0),
        x,
        jnp.concatenate([x[1:], rl], axis=0),
        jnp.concatenate([x[2:], rl, rl], axis=0),
    )
    m5 = (sh[0] + sh[1] + sh[2] + sh[3] + sh[4]) * 0.2
    var5 = ((sh[0] - m5) ** 2 + (sh[1] - m5) ** 2 + (sh[2] - m5) ** 2
            + (sh[3] - m5) ** 2 + (sh[4] - m5) ** 2) * 0.25
    p.append(jnp.sum(var5, axis=-1, keepdims=True))
    part = jnp.concatenate(p, axis=1)

    dif = x[1:] - x[:-1]
    dmean = jnp.mean(dif, axis=0, keepdims=True)
    fqp = jnp.reshape(jnp.sum((dif - dmean) ** 2) / (S - 2), (1, 1))

    @pl.when(c == 0)
    def _():
        st_ref[0] = part
        fq_ref[0] = fqp

    @pl.when(c != 0)
    def _():
        st_ref[0] = st_ref[0] + part
        fq_ref[0] = fq_ref[0] + fqp


def _stats_post_kernel(st_ref, fq_ref, lw_ref, fw_ref, kp_ref, gc_ref,
                       kpany_ref):
    st = st_ref[0]
    inv_d = 1.0 / D
    d1 = st[:, 0:1] * inv_d
    d2 = st[:, 1:2] * inv_d
    d3 = st[:, 2:3] * inv_d
    d4 = st[:, 3:4] * inv_d
    d5 = st[:, 4:5] * inv_d
    tr = st[:, 5:6] * inv_d
    mag = jnp.sqrt(st[:, 6:7])
    lv = st[:, 7:8] * inv_d
    freq = fq_ref[0, 0, 0] * inv_d

    def norm01(v):
        mn = jnp.min(v)
        mx = jnp.max(v)
        return (v - mn) / (mx - mn + 1e-6)

    imp = norm01(0.5 * d1 + 0.3 * d2 + 0.2 * d4)
    lw_ref[0] = jnp.clip(jnp.round(LOCAL_BASE * (0.5 + 0.5 * imp)), 2,
                         min(S, 2 * LOCAL_BASE)).astype(jnp.int32)

    ti = norm01(tr)
    fw_ref[0] = jnp.clip(jnp.round(FUTURE_BASE * (0.5 + 0.5 * ti)), 1,
                         min(S // 2, FUTURE_BASE)).astype(jnp.int32)

    dmm = 0.4 * d1 + 0.3 * d2 + 0.2 * d3 + 0.1 * d5
    mu = jnp.mean(dmm)
    sd = jnp.sqrt(jnp.sum((dmm - mu) ** 2) / (S - 1))
    thr = mu + THRESH * sd
    gt = (dmm > thr).astype(jnp.int32)
    interior = ((dmm[1:-1] > dmm[:-2]) & (dmm[1:-1] > dmm[2:])).astype(
        jnp.int32) * gt[1:-1]
    kp = jnp.concatenate([gt[0:1], interior, gt[-1:]], axis=0)
    kp_ref[0] = kp
    kpany_ref[0] = jnp.max(kp, keepdims=True)

    fs = lv / (freq + 1e-6)
    sidx = jax.lax.broadcasted_iota(jnp.int32, (S, 1), 0)
    fs = jnp.where((sidx > 0) & (sidx < S - 1), fs, 0.0)
    imp2 = norm01(0.3 * mag + 0.4 * d1 + 0.3 * fs)

    seglen = S // 4
    iota_seg = jax.lax.broadcasted_iota(jnp.int32, (seglen, 1), 0)
    for si in range(4):
        seg = imp2[si * seglen:(si + 1) * seglen]
        taken = jnp.zeros((seglen, 1), jnp.bool_)
        for _ in range(4):
            cur = jnp.where(taken, -1e30, seg)
            mx = jnp.max(cur)
            ismx = (cur == mx) & (~taken)
            idx = jnp.min(jnp.where(ismx, iota_seg, seglen))
            taken = taken | (iota_seg == idx)
        gc_ref[0, si * seglen:(si + 1) * seglen] = taken.astype(jnp.int32)


def _proj_kernel(xq_ref, xk_ref, xv_ref,
                 Wq_ref, bq_ref, Wk_ref, bk_ref, Wv_ref, bv_ref,
                 Wg_ref, bg_ref,
                 gq_ref, bqn_ref, gk_ref, bkn_ref, gv_ref, bvn_ref,
                 qo_ref, ko_ref, vo_ref, go_ref):
    def ln(x, g, b):
        mu = jnp.mean(x, axis=-1, keepdims=True)
        var = jnp.mean((x - mu) ** 2, axis=-1, keepdims=True)
        return (x - mu) / jnp.sqrt(var + 1e-5) * g + b

    def matmul_t(x, w_ref, b):
        return jax.lax.dot_general(
            x, w_ref[...], (((1,), (1,)), ((), ())),
            preferred_element_type=jnp.float32) + b

    xq = xq_ref[0]
    qo_ref[0] = matmul_t(ln(xq, gq_ref[...], bqn_ref[...]), Wq_ref, bq_ref[...])
    ko_ref[0] = matmul_t(ln(xk_ref[0], gk_ref[...], bkn_ref[...]), Wk_ref,
                         bk_ref[...])
    vo_ref[0] = matmul_t(ln(xv_ref[0], gv_ref[...], bvn_ref[...]), Wv_ref,
                         bv_ref[...])
    go_ref[0] = jax.nn.sigmoid(matmul_t(xq, Wg_ref, bg_ref[...]))


def _attn_kernel(q_ref, k_ref, v_ref, g_ref,
                 lw_ref, fw_ref, kpr_ref, kpc_ref, gcc_ref, kpany_ref,
                 Wo_ref, bo_ref, o_ref, s_scr, ctx_scr):
    qi = pl.program_id(1)
    qblk = q_ref[0]
    kblk = k_ref[0]
    vblk = v_ref[0]

    icol = qi * BQ + jax.lax.broadcasted_iota(jnp.int32, (BQ, 1), 0)
    jrow = jax.lax.broadcasted_iota(jnp.int32, (1, S), 1)
    lwr = lw_ref[0]
    fwr = fw_ref[0]
    kpr = kpr_ref[0] > 0
    kpc = kpc_ref[0] > 0
    gcc = gcc_ref[0] > 0
    kpany = kpany_ref[0, 0, 0] > 0
    fb = ((jrow == 0) | (jrow == (S - 1) // 4) | (jrow == (S - 1) // 2)
          | (jrow == (3 * (S - 1)) // 4) | (jrow == S - 1))
    win = (jrow >= icol - lwr) & (jrow <= icol + fwr)
    colterm = (kpany & kpc) | ((~kpany) & fb) | gcc
    rowterm = kpany & kpr
    mask = (win | rowterm) | colterm
    negbias = jnp.where(mask, 0.0, NEG)

    for h in range(H):
        qh = qblk[:, h * DH:(h + 1) * DH]
        kh = kblk[:, h * DH:(h + 1) * DH]
        vh = vblk[:, h * DH:(h + 1) * DH]
        s_scr[...] = jax.lax.dot_general(
            qh, kh, (((1,), (1,)), ((), ())),
            preferred_element_type=jnp.float32) * 0.125 + negbias
        p = jnp.exp(s_scr[...] - 40.0)
        inv = 1.0 / jnp.sum(p, axis=-1, keepdims=True)
        ctx_scr[:, h * DH:(h + 1) * DH] = jnp.dot(
            p, vh, preferred_element_type=jnp.float32) * inv
    ctxg = ctx_scr[...] * g_ref[0]
    o_ref[0] = jax.lax.dot_general(
        ctxg, Wo_ref[...], (((1,), (1,)), ((), ())),
        preferred_element_type=jnp.float32) + bo_ref[...]


def _mask_ingredients(queries):
    f32 = jnp.float32
    i32 = jnp.int32

    stats, fq = pl.pallas_call(
        _stats_part_kernel,
        grid=(B, NC),
        in_specs=[pl.BlockSpec((1, S, CW), lambda b, c: (b, 0, c))],
        out_specs=[
            pl.BlockSpec((1, S, 8), lambda b, c: (b, 0, 0)),
            pl.BlockSpec((1, 1, 1), lambda b, c: (b, 0, 0)),
        ],
        out_shape=[
            jax.ShapeDtypeStruct((B, S, 8), f32),
            jax.ShapeDtypeStruct((B, 1, 1), f32),
        ],
    )(queries)

    lw, fw, kp, gc, kpany = pl.pallas_call(
        _stats_post_kernel,
        grid=(B,),
        in_specs=[pl.BlockSpec((1, S, 8), lambda b: (b, 0, 0)),
                  pl.BlockSpec((1, 1, 1), lambda b: (b, 0, 0))],
        out_specs=[
            pl.BlockSpec((1, S, 1), lambda b: (b, 0, 0)),
            pl.BlockSpec((1, S, 1), lambda b: (b, 0, 0)),
            pl.BlockSpec((1, S, 1), lambda b: (b, 0, 0)),
            pl.BlockSpec((1, S, 1), lambda b: (b, 0, 0)),
            pl.BlockSpec((1, 1, 1), lambda b: (b, 0, 0)),
        ],
        out_shape=[
            jax.ShapeDtypeStruct((B, S, 1), i32),
            jax.ShapeDtypeStruct((B, S, 1), i32),
            jax.ShapeDtypeStruct((B, S, 1), i32),
            jax.ShapeDtypeStruct((B, S, 1), i32),
            jax.ShapeDtypeStruct((B, 1, 1), i32),
        ],
    )(stats, fq)
    return lw, fw, kp, gc, kpany


def kernel(queries, keys, values, Wq, bq, Wk, bk, Wv, bv, Wo, bo, Wg, bg,
           g_q, b_q, g_k, b_k, g_v, b_v):
    f32 = jnp.float32

    lw, fw, kp, gc, kpany = _mask_ingredients(queries)
    kp_col = kp.reshape(B, 1, S)
    gc_col = gc.reshape(B, 1, S)

    b2 = lambda a: a.reshape(1, D)
    nrow = S // BS
    wspec = pl.BlockSpec((D, D), lambda b, r: (0, 0))
    bspec = pl.BlockSpec((1, D), lambda b, r: (0, 0))
    xspec = pl.BlockSpec((1, BS, D), lambda b, r: (b, r, 0))
    q, k, v, gate = pl.pallas_call(
        _proj_kernel,
        grid=(B, nrow),
        in_specs=[xspec, xspec, xspec,
                  wspec, bspec, wspec, bspec, wspec, bspec, wspec, bspec,
                  bspec, bspec, bspec, bspec, bspec, bspec],
        out_specs=[xspec, xspec, xspec, xspec],
        out_shape=[jax.ShapeDtypeStruct((B, S, D), f32)] * 4,
    )(queries, keys, values,
      Wq, b2(bq), Wk, b2(bk), Wv, b2(bv), Wg, b2(bg),
      b2(g_q), b2(b_q), b2(g_k), b2(b_k), b2(g_v), b2(b_v))

    nq = S // BQ
    qspec = pl.BlockSpec((1, BQ, D), lambda b, r: (b, r, 0))
    fullspec = pl.BlockSpec((1, S, D), lambda b, r: (b, 0, 0))
    rowspec = pl.BlockSpec((1, BQ, 1), lambda b, r: (b, r, 0))
    colspec = pl.BlockSpec((1, 1, S), lambda b, r: (b, 0, 0))
    out = pl.pallas_call(
        _attn_kernel,
        grid=(B, nq),
        in_specs=[qspec, fullspec, fullspec, qspec,
                  rowspec, rowspec, rowspec, colspec, colspec,
                  pl.BlockSpec((1, 1, 1), lambda b, r: (b, 0, 0)),
                  pl.BlockSpec((D, D), lambda b, r: (0, 0)),
                  pl.BlockSpec((1, D), lambda b, r: (0, 0))],
        out_specs=qspec,
        out_shape=jax.ShapeDtypeStruct((B, S, D), f32),
        scratch_shapes=[pltpu.VMEM((BQ, S), f32), pltpu.VMEM((BQ, D), f32)],
        compiler_params=pltpu.CompilerParams(
            vmem_limit_bytes=64 * 1024 * 1024),
    )(q, k, v, gate, lw, fw, kp, kp_col, gc_col, kpany, Wo, b2(bo))
    return out
